# Initial kernel scaffold; baseline (speedup 1.0000x reference)
import functools

import jax
import jax.numpy as jnp
from jax import lax
from jax.experimental import pallas as pl
from jax.experimental.pallas import tpu as pltpu
from jax.experimental.pallas import tpu_sc as plsc

N = 10000
E = 320000
D = 128
NUM_LAYERS = 3

NP = 10240
NC = 2
NS = 16
NTILES = NC * NS
CH = 128
NCH = (E // NTILES + CH - 1) // CH
ET = NCH * CH
EP = NTILES * ET
RPT = NP // NS

_f32 = jnp.float32


_BR = 2560


def _prep0_body(x_ref, w_ref, a2_ref, h_ref, p_ref):
    h = jnp.dot(x_ref[...], w_ref[...], preferred_element_type=_f32)
    h_ref[...] = h
    p_ref[...] = jnp.dot(h, a2_ref[...], preferred_element_type=_f32)


def _prep0(x, W, A2):
    return pl.pallas_call(
        _prep0_body,
        grid=(NP // _BR,),
        in_specs=[
            pl.BlockSpec((_BR, D), lambda i: (i, 0)),
            pl.BlockSpec((D, D), lambda i: (0, 0)),
            pl.BlockSpec((D, D), lambda i: (0, 0)),
        ],
        out_specs=[
            pl.BlockSpec((_BR, D), lambda i: (i, 0)),
            pl.BlockSpec((_BR, D), lambda i: (i, 0)),
        ],
        out_shape=[
            jax.ShapeDtypeStruct((NP, D), _f32),
            jax.ShapeDtypeStruct((NP, D), _f32),
        ],
    )(x, W, A2)


def _prepl_body(acc_ref, d_ref, b_ref, w_ref, a2_ref, x_ref, h_ref, p_ref):
    x = (acc_ref[0] + acc_ref[1]) / (d_ref[...] + 1e-16) + b_ref[...]
    x_ref[...] = x
    h = jnp.dot(x, w_ref[...], preferred_element_type=_f32)
    h_ref[...] = h
    p_ref[...] = jnp.dot(h, a2_ref[...], preferred_element_type=_f32)


def _prepl(acc, dsum, b1, W, A2):
    return pl.pallas_call(
        _prepl_body,
        grid=(NP // _BR,),
        in_specs=[
            pl.BlockSpec((NC, _BR, D), lambda i: (0, i, 0)),
            pl.BlockSpec((_BR, 1), lambda i: (i, 0)),
            pl.BlockSpec((1, D), lambda i: (0, 0)),
            pl.BlockSpec((D, D), lambda i: (0, 0)),
            pl.BlockSpec((D, D), lambda i: (0, 0)),
        ],
        out_specs=[
            pl.BlockSpec((_BR, D), lambda i: (i, 0)),
            pl.BlockSpec((_BR, D), lambda i: (i, 0)),
            pl.BlockSpec((_BR, D), lambda i: (i, 0)),
        ],
        out_shape=[
            jax.ShapeDtypeStruct((NP, D), _f32),
            jax.ShapeDtypeStruct((NP, D), _f32),
            jax.ShapeDtypeStruct((NP, D), _f32),
        ],
    )(acc, dsum, b1, W, A2)


def _denmerge_body(d_ref, o_ref):
    o_ref[...] = jnp.sum(d_ref[...], axis=0)


def _denmerge(den):
    return pl.pallas_call(
        _denmerge_body,
        out_shape=jax.ShapeDtypeStruct((NP // 128, 128), _f32),
    )(den)


def _final_body(x0_ref, x1_ref, x2_ref, acc_ref, d_ref, b_ref, wo_ref,
                bo_ref, y_ref):
    x3 = (acc_ref[0] + acc_ref[1]) / (d_ref[...] + 1e-16) + b_ref[...]
    y = jnp.dot(x0_ref[...], wo_ref[0], preferred_element_type=_f32)
    y += jnp.dot(x1_ref[...], wo_ref[1], preferred_element_type=_f32)
    y += jnp.dot(x2_ref[...], wo_ref[2], preferred_element_type=_f32)
    y += jnp.dot(x3, wo_ref[3], preferred_element_type=_f32)
    y_ref[...] = y + bo_ref[...]


def _final(x0, x1, x2, acc, dsum, b1, Wo, bo1):
    return pl.pallas_call(
        _final_body,
        grid=(NP // _BR,),
        in_specs=[
            pl.BlockSpec((_BR, D), lambda i: (i, 0)),
            pl.BlockSpec((_BR, D), lambda i: (i, 0)),
            pl.BlockSpec((_BR, D), lambda i: (i, 0)),
            pl.BlockSpec((NC, _BR, D), lambda i: (0, i, 0)),
            pl.BlockSpec((_BR, 1), lambda i: (i, 0)),
            pl.BlockSpec((1, D), lambda i: (0, 0)),
            pl.BlockSpec((4, D, D), lambda i: (0, 0, 0)),
            pl.BlockSpec((1, D), lambda i: (0, 0)),
        ],
        out_specs=pl.BlockSpec((_BR, D), lambda i: (i, 0)),
        out_shape=jax.ShapeDtypeStruct((NP, D), _f32),
    )(x0, x1, x2, acc, dsum, b1, Wo, bo1)


def _sc_edge_body(h_hbm, sa_hbm, ad_hbm, src_hbm, dst_hbm,
                  acc_hbm, den_hbm,
                  acc_s, sa_t, ad_t, src_t, dst_t, exs, rows, denl,
                  gsem, ssem):
    c = lax.axis_index("c")
    s = lax.axis_index("s")
    tile = c * NS + s

    pltpu.sync_copy(sa_hbm, sa_t)
    pltpu.sync_copy(ad_hbm, ad_t)
    pltpu.sync_copy(src_hbm.at[tile], src_t)
    pltpu.sync_copy(dst_hbm.at[tile], dst_t)

    zv = jnp.zeros((16,), _f32)

    def _zero_denl(i, _):
        denl[pl.ds(i * 16, 16)] = zv
        return 0

    lax.fori_loop(0, NP // 16, _zero_denl, 0)

    def _zero_rows(i, _):
        for j in range(8):
            rows[i, pl.ds(j * 16, 16)] = zv
        return 0

    lax.fori_loop(0, CH, _zero_rows, 0)

    for k in range(RPT // CH):
        pltpu.sync_copy(rows, acc_s.at[pl.ds((s * (RPT // CH) + k) * CH, CH)])

    def _mx(i, acc_v):
        return jnp.maximum(acc_v, sa_t[pl.ds(i * 16, 16)])

    mv = lax.fori_loop(0, NP // 16, _mx, jnp.full((16,), -3.0e38, _f32))
    max_s = jnp.max(mv)

    plsc.subcore_barrier()

    def _chunk(ch, _):
        gcp = pltpu.async_copy(h_hbm.at[src_t.at[ch]], rows, gsem)

        for g in range(8):
            si = src_t[ch, pl.ds(g * 16, 16)]
            di = dst_t[ch, pl.ds(g * 16, 16)]
            a1 = plsc.load_gather(sa_t, [si])
            a2 = plsc.load_gather(ad_t, [di])
            z = a1 + a2
            e = jnp.where(z >= 0.0, z, 0.2 * z)
            m = jnp.maximum(a2 + max_s, 0.0)
            ex = jnp.exp(e - m)
            exs[pl.ds(g * 16, 16)] = ex
            plsc.addupdate_scatter(denl, [di], ex)

        gcp.wait()

        def _scale(i, _):
            ev = plsc.load_gather(exs, [jnp.full((16,), i, jnp.int32)])
            for j in range(8):
                rows[i, pl.ds(j * 16, 16)] = rows[i, pl.ds(j * 16, 16)] * ev
            return 0

        lax.fori_loop(0, CH, _scale, 0)

        pltpu.async_copy(rows, acc_s.at[dst_t.at[ch]], ssem, add=True).wait()
        return 0

    lax.fori_loop(0, NCH, _chunk, 0)

    pltpu.sync_copy(denl, den_hbm.at[c, s])
    plsc.subcore_barrier()

    pltpu.sync_copy(acc_s.at[pl.ds(s * RPT, RPT)],
                    acc_hbm.at[c, pl.ds(s * RPT, RPT)])


_sc_edge = pl.kernel(
    _sc_edge_body,
    out_type=[
        jax.ShapeDtypeStruct((NC, NP, D), _f32),
        jax.ShapeDtypeStruct((NC, NS, NP), _f32),
    ],
    mesh=plsc.VectorSubcoreMesh(core_axis_name="c", subcore_axis_name="s"),
    scratch_types=[
        pltpu.VMEM_SHARED((NP, D), _f32),
        pltpu.VMEM((NP,), _f32),
        pltpu.VMEM((NP,), _f32),
        pltpu.VMEM((NCH, CH), jnp.int32),
        pltpu.VMEM((NCH, CH), jnp.int32),
        pltpu.VMEM((CH,), _f32),
        pltpu.VMEM((CH, D), _f32),
        pltpu.VMEM((NP,), _f32),
        pltpu.SemaphoreType.DMA,
        pltpu.SemaphoreType.DMA,
    ],
)


def kernel(features, edge_index, W, a_src, a_dst, b, W_out, b_out):
    xp = jnp.pad(features, ((0, NP - N), (0, 0)))
    src = edge_index[0]
    dst = edge_index[1]
    srcp = jnp.concatenate(
        [src, jnp.zeros((EP - E,), jnp.int32)]).reshape(NTILES, NCH, CH)
    dstp = jnp.concatenate(
        [dst, jnp.full((EP - E,), N, jnp.int32)]).reshape(NTILES, NCH, CH)
    A2 = jnp.zeros((D, D), _f32).at[:, 0].set(a_src).at[:, 1].set(a_dst)
    b1 = b.reshape(1, D)
    bo1 = b_out.reshape(1, D)
    Wo = W_out.reshape(4, D, D)

    h, P = _prep0(xp, W, A2)
    xs = [xp]
    y = None
    for layer in range(NUM_LAYERS):
        sa = P[:, 0]
        ad = P[:, 1]
        acc, den = _sc_edge(h, sa, ad, srcp, dstp)
        dsum = _denmerge(den.reshape(NTILES, NP // 128, 128)).reshape(NP, 1)
        if layer < NUM_LAYERS - 1:
            x, h, P = _prepl(acc, dsum, b1, W, A2)
            xs.append(x)
        else:
            y = _final(xs[0], xs[1], xs[2], acc, dsum, b1, Wo, bo1)
    return y[:N]


# speedup vs baseline: 15.6080x; 15.6080x over previous
import functools

import jax
import jax.numpy as jnp
from jax import lax
from jax.experimental import pallas as pl
from jax.experimental.pallas import tpu as pltpu
from jax.experimental.pallas import tpu_sc as plsc

N = 10000
E = 320000
D = 128
NUM_LAYERS = 3

NP = 10240
NC = 2
NS = 16
NTILES = NC * NS
CH = 128
NCH = (E // NTILES + CH - 1) // CH
ET = NCH * CH
EP = NTILES * ET
RPT = NP // NS

_f32 = jnp.float32


_BR = 2560


def _prep0_body(x_ref, w_ref, a2_ref, h_ref, p_ref):
    h = jnp.dot(x_ref[...], w_ref[...], preferred_element_type=_f32)
    h_ref[...] = h
    p_ref[...] = jnp.dot(h, a2_ref[...], preferred_element_type=_f32)


def _prep0(x, W, A2):
    return pl.pallas_call(
        _prep0_body,
        grid=(NP // _BR,),
        in_specs=[
            pl.BlockSpec((_BR, D), lambda i: (i, 0)),
            pl.BlockSpec((D, D), lambda i: (0, 0)),
            pl.BlockSpec((D, D), lambda i: (0, 0)),
        ],
        out_specs=[
            pl.BlockSpec((_BR, D), lambda i: (i, 0)),
            pl.BlockSpec((_BR, D), lambda i: (i, 0)),
        ],
        out_shape=[
            jax.ShapeDtypeStruct((NP, D), _f32),
            jax.ShapeDtypeStruct((NP, D), _f32),
        ],
    )(x, W, A2)


def _prepl_body(acc_ref, d_ref, b_ref, w_ref, a2_ref, x_ref, h_ref, p_ref):
    x = (acc_ref[0] + acc_ref[1]) / (d_ref[...] + 1e-16) + b_ref[...]
    x_ref[...] = x
    h = jnp.dot(x, w_ref[...], preferred_element_type=_f32)
    h_ref[...] = h
    p_ref[...] = jnp.dot(h, a2_ref[...], preferred_element_type=_f32)


def _prepl(acc, dsum, b1, W, A2):
    return pl.pallas_call(
        _prepl_body,
        grid=(NP // _BR,),
        in_specs=[
            pl.BlockSpec((NC, _BR, D), lambda i: (0, i, 0)),
            pl.BlockSpec((_BR, 1), lambda i: (i, 0)),
            pl.BlockSpec((1, D), lambda i: (0, 0)),
            pl.BlockSpec((D, D), lambda i: (0, 0)),
            pl.BlockSpec((D, D), lambda i: (0, 0)),
        ],
        out_specs=[
            pl.BlockSpec((_BR, D), lambda i: (i, 0)),
            pl.BlockSpec((_BR, D), lambda i: (i, 0)),
            pl.BlockSpec((_BR, D), lambda i: (i, 0)),
        ],
        out_shape=[
            jax.ShapeDtypeStruct((NP, D), _f32),
            jax.ShapeDtypeStruct((NP, D), _f32),
            jax.ShapeDtypeStruct((NP, D), _f32),
        ],
    )(acc, dsum, b1, W, A2)


def _maxs_body(p_ref, o_ref):
    o_ref[...] = jnp.full((1, 16), jnp.max(p_ref[...][:, 0:1]), _f32)


def _maxs(P):
    return pl.pallas_call(
        _maxs_body,
        out_shape=jax.ShapeDtypeStruct((1, 16), _f32),
    )(P)


def _denmerge_body(d_ref, o_ref):
    o_ref[...] = jnp.sum(d_ref[...], axis=0)


def _denmerge(den):
    return pl.pallas_call(
        _denmerge_body,
        out_shape=jax.ShapeDtypeStruct((NP // 128, 128), _f32),
    )(den)


def _final_body(x0_ref, x1_ref, x2_ref, acc_ref, d_ref, b_ref, wo_ref,
                bo_ref, y_ref):
    x3 = (acc_ref[0] + acc_ref[1]) / (d_ref[...] + 1e-16) + b_ref[...]
    y = jnp.dot(x0_ref[...], wo_ref[0], preferred_element_type=_f32)
    y += jnp.dot(x1_ref[...], wo_ref[1], preferred_element_type=_f32)
    y += jnp.dot(x2_ref[...], wo_ref[2], preferred_element_type=_f32)
    y += jnp.dot(x3, wo_ref[3], preferred_element_type=_f32)
    y_ref[...] = y + bo_ref[...]


def _final(x0, x1, x2, acc, dsum, b1, Wo, bo1):
    return pl.pallas_call(
        _final_body,
        grid=(NP // _BR,),
        in_specs=[
            pl.BlockSpec((_BR, D), lambda i: (i, 0)),
            pl.BlockSpec((_BR, D), lambda i: (i, 0)),
            pl.BlockSpec((_BR, D), lambda i: (i, 0)),
            pl.BlockSpec((NC, _BR, D), lambda i: (0, i, 0)),
            pl.BlockSpec((_BR, 1), lambda i: (i, 0)),
            pl.BlockSpec((1, D), lambda i: (0, 0)),
            pl.BlockSpec((4, D, D), lambda i: (0, 0, 0)),
            pl.BlockSpec((1, D), lambda i: (0, 0)),
        ],
        out_specs=pl.BlockSpec((_BR, D), lambda i: (i, 0)),
        out_shape=jax.ShapeDtypeStruct((NP, D), _f32),
    )(x0, x1, x2, acc, dsum, b1, Wo, bo1)


def _sc_edge_body(h_hbm, sa_hbm, ad_hbm, mx_hbm, src_hbm, dst_hbm,
                  acc_hbm, den_hbm,
                  acc_s, den_s, sa_t, ad_t, mx_t, src_t, dst_t, exs, rows,
                  zbuf, gsem, ssem, isem):
    c = lax.axis_index("c")
    s = lax.axis_index("s")
    tile = c * NS + s

    pltpu.sync_copy(sa_hbm, sa_t)
    pltpu.sync_copy(ad_hbm, ad_t)
    pltpu.sync_copy(mx_hbm, mx_t)

    zv = jnp.zeros((16,), _f32)

    def _zero_rows(i, _):
        for j in range(8):
            rows[i, pl.ds(j * 16, 16)] = zv
        return 0

    lax.fori_loop(0, CH, _zero_rows, 0)

    def _zero_zbuf(i, _):
        zbuf[pl.ds(i * 16, 16)] = zv
        return 0

    lax.fori_loop(0, RPT // 16, _zero_zbuf, 0)

    for k in range(RPT // CH):
        pltpu.sync_copy(rows, acc_s.at[pl.ds((s * (RPT // CH) + k) * CH, CH)])
    pltpu.sync_copy(zbuf, den_s.at[pl.ds(s * RPT, RPT)])

    max_s = mx_t[...]

    plsc.subcore_barrier()

    def _chunk(ch, _):
        pltpu.sync_copy(src_hbm.at[tile, ch], src_t)
        pltpu.sync_copy(dst_hbm.at[tile, ch], dst_t)
        gcp = pltpu.async_copy(h_hbm.at[src_t], rows, gsem)

        for g in range(8):
            si = src_t[pl.ds(g * 16, 16)]
            di = dst_t[pl.ds(g * 16, 16)]
            a1 = plsc.load_gather(sa_t, [si])
            a2 = plsc.load_gather(ad_t, [di])
            z = a1 + a2
            e = jnp.where(z >= 0.0, z, 0.2 * z)
            m = jnp.maximum(a2 + max_s, 0.0)
            ex = jnp.exp(e - m)
            exs[pl.ds(g * 16, 16)] = ex

        dcp = pltpu.async_copy(exs, den_s.at[dst_t], isem, add=True)

        gcp.wait()

        def _scale(i, _):
            ev = plsc.load_gather(exs, [jnp.full((16,), i, jnp.int32)])
            for j in range(8):
                rows[i, pl.ds(j * 16, 16)] = rows[i, pl.ds(j * 16, 16)] * ev
            return 0

        lax.fori_loop(0, CH, _scale, 0)

        pltpu.async_copy(rows, acc_s.at[dst_t], ssem, add=True).wait()
        dcp.wait()
        return 0

    lax.fori_loop(0, NCH, _chunk, 0)

    plsc.subcore_barrier()

    pltpu.sync_copy(acc_s.at[pl.ds(s * RPT, RPT)],
                    acc_hbm.at[c, pl.ds(s * RPT, RPT)])
    pltpu.sync_copy(den_s.at[pl.ds(s * RPT, RPT)],
                    den_hbm.at[c, pl.ds(s * RPT, RPT)])


_sc_edge = pl.kernel(
    _sc_edge_body,
    out_type=[
        jax.ShapeDtypeStruct((NC, NP, D), _f32),
        jax.ShapeDtypeStruct((NC, NP), _f32),
    ],
    mesh=plsc.VectorSubcoreMesh(core_axis_name="c", subcore_axis_name="s"),
    compiler_params=pltpu.CompilerParams(needs_layout_passes=False),
    scratch_types=[
        pltpu.VMEM_SHARED((NP, D), _f32),
        pltpu.VMEM_SHARED((NP,), _f32),
        pltpu.VMEM((NP,), _f32),
        pltpu.VMEM((NP,), _f32),
        pltpu.VMEM((16,), _f32),
        pltpu.VMEM((CH,), jnp.int32),
        pltpu.VMEM((CH,), jnp.int32),
        pltpu.VMEM((CH,), _f32),
        pltpu.VMEM((CH, D), _f32),
        pltpu.VMEM((RPT,), _f32),
        pltpu.SemaphoreType.DMA,
        pltpu.SemaphoreType.DMA,
        pltpu.SemaphoreType.DMA,
    ],
)


def kernel(features, edge_index, W, a_src, a_dst, b, W_out, b_out):
    xp = jnp.pad(features, ((0, NP - N), (0, 0)))
    src = edge_index[0]
    dst = edge_index[1]
    srcp = jnp.concatenate(
        [src, jnp.zeros((EP - E,), jnp.int32)]).reshape(NTILES, NCH, CH)
    dstp = jnp.concatenate(
        [dst, jnp.full((EP - E,), N, jnp.int32)]).reshape(NTILES, NCH, CH)
    A2 = jnp.zeros((D, D), _f32).at[:, 0].set(a_src).at[:, 1].set(a_dst)
    b1 = b.reshape(1, D)
    bo1 = b_out.reshape(1, D)
    Wo = W_out.reshape(4, D, D)

    h, P = _prep0(xp, W, A2)
    xs = [xp]
    y = None
    for layer in range(NUM_LAYERS):
        sa = P[:, 0]
        ad = P[:, 1]
        mx16 = _maxs(P).reshape(16)
        acc, den = _sc_edge(h, sa, ad, mx16, srcp, dstp)
        dsum = _denmerge(den.reshape(NC, NP // 128, 128)).reshape(NP, 1)
        if layer < NUM_LAYERS - 1:
            x, h, P = _prepl(acc, dsum, b1, W, A2)
            xs.append(x)
        else:
            y = _final(xs[0], xs[1], xs[2], acc, dsum, b1, Wo, bo1)
    return y[:N]
